# baseline (speedup 1.0000x reference)
import functools

import jax
import jax.numpy as jnp
from jax.experimental import pallas as pl
from jax.experimental.pallas import tpu as pltpu

NE = 16
TOPK = 8


def _router_body(x_ref, gw_ref, gb_ref, probs_ref, idx_ref, comb_ref, xbf_ref):
    x = x_ref[...]
    xbf = x.astype(jnp.bfloat16)
    logits = jax.lax.dot_general(
        xbf, gw_ref[...].astype(jnp.bfloat16), (((1,), (1,)), ((), ())),
        preferred_element_type=jnp.float32,
    )
    logits = logits + gb_ref[...]
    m = jnp.max(logits, axis=1, keepdims=True)
    ex = jnp.exp(logits - m)
    p = ex / jnp.sum(ex, axis=1, keepdims=True)

    bt = p.shape[0]
    lane = jax.lax.broadcasted_iota(jnp.int32, (bt, NE), 1)
    work = p
    sel = jnp.zeros((bt, NE), jnp.bool_)
    idx_cols = []
    for _ in range(TOPK):
        mx = jnp.max(work, axis=1, keepdims=True)
        kidx = jnp.min(jnp.where(work == mx, lane, NE), axis=1)
        idx_cols.append(kidx[:, None])
        hit = lane == kidx[:, None]
        sel = sel | hit
        work = jnp.where(hit, -1.0, work)
    selp = jnp.where(sel, p, 0.0)
    comb_ref[...] = selp / jnp.sum(selp, axis=1, keepdims=True)
    probs_ref[...] = p
    idx_ref[...] = jnp.concatenate(idx_cols, axis=1)
    xbf_ref[...] = xbf


def _expert_body(xbf_ref, w_ref, comb_ref, eb_ref, out_ref):
    q = pl.program_id(1)
    t = xbf_ref.shape[0]
    ws = [w_ref[i].astype(jnp.bfloat16) for i in range(8)]

    ct = 512
    for tb in range(t // ct):
        sl = pl.ds(tb * ct, ct)
        x = xbf_ref[sl, :]
        comb = comb_ref[sl, :]
        lane = jax.lax.broadcasted_iota(jnp.int32, (ct, NE), 1)
        upd = None
        for i in range(8):
            ci = jnp.sum(jnp.where(lane == 8 * q + i, comb, 0.0), axis=1,
                         keepdims=True)
            yi = jax.lax.dot_general(
                x, ws[i], (((1,), (1,)), ((), ())),
                preferred_element_type=jnp.float32,
            )
            upd = ci * yi if upd is None else upd + ci * yi

        @pl.when(q == 0)
        def _init(upd=upd, comb=comb, sl=sl):
            bias = jax.lax.dot_general(
                comb, eb_ref[...], (((1,), (0,)), ((), ())),
                precision=jax.lax.Precision.HIGHEST,
                preferred_element_type=jnp.float32,
            )
            out_ref[sl, :] = bias + upd

        @pl.when(q != 0)
        def _acc(upd=upd, sl=sl):
            out_ref[sl, :] += upd


def kernel(x, gate_w, gate_b, expert_w, expert_b):
    bsz, seq, h = x.shape
    t = bsz * seq
    xf = x.reshape(t, h)

    bt = 512
    router = pl.pallas_call(
        _router_body,
        grid=(t // bt,),
        in_specs=[
            pl.BlockSpec((bt, h), lambda i: (i, 0)),
            pl.BlockSpec((NE, h), lambda i: (0, 0)),
            pl.BlockSpec((1, NE), lambda i: (0, 0)),
        ],
        out_specs=[
            pl.BlockSpec((bt, NE), lambda i: (i, 0)),
            pl.BlockSpec((bt, TOPK), lambda i: (i, 0)),
            pl.BlockSpec((bt, NE), lambda i: (i, 0)),
            pl.BlockSpec((bt, h), lambda i: (i, 0)),
        ],
        out_shape=[
            jax.ShapeDtypeStruct((t, NE), jnp.float32),
            jax.ShapeDtypeStruct((t, TOPK), jnp.int32),
            jax.ShapeDtypeStruct((t, NE), jnp.float32),
            jax.ShapeDtypeStruct((t, h), jnp.bfloat16),
        ],
    )
    probs, topk_idx, combine, x_bf = router(xf, gate_w, gate_b.reshape(1, NE))

    bh = 128
    experts = pl.pallas_call(
        _expert_body,
        grid=(h // bh, NE // 8),
        in_specs=[
            pl.BlockSpec((t, h), lambda j, p: (0, 0)),
            pl.BlockSpec((8, bh, h), lambda j, p: (p, j, 0)),
            pl.BlockSpec((t, NE), lambda j, p: (0, 0)),
            pl.BlockSpec((NE, bh), lambda j, p: (0, j)),
        ],
        out_specs=pl.BlockSpec((t, bh), lambda j, p: (0, j)),
        out_shape=jax.ShapeDtypeStruct((t, h), jnp.float32),
        compiler_params=pltpu.CompilerParams(
            dimension_semantics=("parallel", "arbitrary"),
        ),
    )
    out = experts(x_bf, expert_w, combine, expert_b)
    return (out.reshape(bsz, seq, h), topk_idx, probs)


# speedup vs baseline: 1.8407x; 1.8407x over previous
import functools

import jax
import jax.numpy as jnp
from jax.experimental import pallas as pl
from jax.experimental.pallas import tpu as pltpu

NE = 16
TOPK = 8


def _router_body(x_ref, gw_ref, gb_ref, probs_ref, idx_ref, comb_ref, xbf_ref):
    x = x_ref[...]
    xbf = x.astype(jnp.bfloat16)
    logits = jax.lax.dot_general(
        xbf, gw_ref[...].astype(jnp.bfloat16), (((1,), (1,)), ((), ())),
        preferred_element_type=jnp.float32,
    )
    logits = logits + gb_ref[...]
    m = jnp.max(logits, axis=1, keepdims=True)
    ex = jnp.exp(logits - m)
    p = ex / jnp.sum(ex, axis=1, keepdims=True)

    bt = p.shape[0]
    lane = jax.lax.broadcasted_iota(jnp.int32, (bt, NE), 1)
    work = p
    sel = jnp.zeros((bt, NE), jnp.bool_)
    idx_cols = []
    for _ in range(TOPK):
        mx = jnp.max(work, axis=1, keepdims=True)
        kidx = jnp.min(jnp.where(work == mx, lane, NE), axis=1)
        idx_cols.append(kidx[:, None])
        hit = lane == kidx[:, None]
        sel = sel | hit
        work = jnp.where(hit, -1.0, work)
    selp = jnp.where(sel, p, 0.0)
    comb_ref[...] = selp / jnp.sum(selp, axis=1, keepdims=True)
    probs_ref[...] = p
    idx_ref[...] = jnp.concatenate(idx_cols, axis=1)
    xbf_ref[...] = xbf


def _expert_body(xbf_ref, w_ref, comb_ref, eb_ref, out_ref):
    q = pl.program_id(1)
    t = xbf_ref.shape[0]
    ws = [w_ref[i].astype(jnp.bfloat16) for i in range(4)]

    ct = 1024
    for tb in range(t // ct):
        sl = pl.ds(tb * ct, ct)
        x = xbf_ref[sl, :]
        comb = comb_ref[sl, :]
        lane = jax.lax.broadcasted_iota(jnp.int32, (ct, NE), 1)
        upd = None
        for i in range(4):
            ci = jnp.sum(jnp.where(lane == 4 * q + i, comb, 0.0), axis=1,
                         keepdims=True)
            yi = jax.lax.dot_general(
                x, ws[i], (((1,), (1,)), ((), ())),
                preferred_element_type=jnp.float32,
            )
            upd = ci * yi if upd is None else upd + ci * yi

        @pl.when(q == 0)
        def _init(upd=upd, comb=comb, sl=sl):
            bias = jax.lax.dot_general(
                comb, eb_ref[...], (((1,), (0,)), ((), ())),
                precision=jax.lax.Precision.HIGHEST,
                preferred_element_type=jnp.float32,
            )
            out_ref[sl, :] = bias + upd

        @pl.when(q != 0)
        def _acc(upd=upd, sl=sl):
            out_ref[sl, :] += upd


def kernel(x, gate_w, gate_b, expert_w, expert_b):
    bsz, seq, h = x.shape
    t = bsz * seq
    xf = x.reshape(t, h)

    bt = 1024
    router = pl.pallas_call(
        _router_body,
        grid=(t // bt,),
        in_specs=[
            pl.BlockSpec((bt, h), lambda i: (i, 0)),
            pl.BlockSpec((NE, h), lambda i: (0, 0)),
            pl.BlockSpec((1, NE), lambda i: (0, 0)),
        ],
        out_specs=[
            pl.BlockSpec((bt, NE), lambda i: (i, 0)),
            pl.BlockSpec((bt, TOPK), lambda i: (i, 0)),
            pl.BlockSpec((bt, NE), lambda i: (i, 0)),
            pl.BlockSpec((bt, h), lambda i: (i, 0)),
        ],
        out_shape=[
            jax.ShapeDtypeStruct((t, NE), jnp.float32),
            jax.ShapeDtypeStruct((t, TOPK), jnp.int32),
            jax.ShapeDtypeStruct((t, NE), jnp.float32),
            jax.ShapeDtypeStruct((t, h), jnp.bfloat16),
        ],
    )
    probs, topk_idx, combine, x_bf = router(xf, gate_w, gate_b.reshape(1, NE))

    bh = 256
    experts = pl.pallas_call(
        _expert_body,
        grid=(h // bh, NE // 4),
        in_specs=[
            pl.BlockSpec((t, h), lambda j, p: (0, 0)),
            pl.BlockSpec((4, bh, h), lambda j, p: (p, j, 0)),
            pl.BlockSpec((t, NE), lambda j, p: (0, 0)),
            pl.BlockSpec((NE, bh), lambda j, p: (0, j)),
        ],
        out_specs=pl.BlockSpec((t, bh), lambda j, p: (0, j)),
        out_shape=jax.ShapeDtypeStruct((t, h), jnp.float32),
        compiler_params=pltpu.CompilerParams(
            dimension_semantics=("parallel", "arbitrary"),
        ),
    )
    out = experts(x_bf, expert_w, combine, expert_b)
    return (out.reshape(bsz, seq, h), topk_idx, probs)


# speedup vs baseline: 1.8415x; 1.0004x over previous
import jax
import jax.numpy as jnp
from jax.experimental import pallas as pl
from jax.experimental.pallas import tpu as pltpu

NE = 16
TOPK = 8


def _router_body(x_ref, gw_ref, gb_ref, probs_ref, idx_ref, comb_ref, xbf_ref):
    x = x_ref[...]
    xbf = x.astype(jnp.bfloat16)
    logits = jax.lax.dot_general(
        xbf, gw_ref[...].astype(jnp.bfloat16), (((1,), (1,)), ((), ())),
        preferred_element_type=jnp.float32,
    )
    logits = logits + gb_ref[...]
    m = jnp.max(logits, axis=1, keepdims=True)
    ex = jnp.exp(logits - m)
    p = ex / jnp.sum(ex, axis=1, keepdims=True)

    bt = p.shape[0]
    lane = jax.lax.broadcasted_iota(jnp.int32, (bt, NE), 1)
    work = p
    sel = jnp.zeros((bt, NE), jnp.bool_)
    idx_cols = []
    for _ in range(TOPK):
        mx = jnp.max(work, axis=1, keepdims=True)
        kidx = jnp.min(jnp.where(work == mx, lane, NE), axis=1)
        idx_cols.append(kidx[:, None])
        hit = lane == kidx[:, None]
        sel = sel | hit
        work = jnp.where(hit, -1.0, work)
    selp = jnp.where(sel, p, 0.0)
    comb_ref[...] = selp / jnp.sum(selp, axis=1, keepdims=True)
    probs_ref[...] = p
    idx_ref[...] = jnp.concatenate(idx_cols, axis=1)
    xbf_ref[...] = xbf


def _expert_body(xbf_ref, w_ref, comb_ref, eb_ref, out_ref):
    q = pl.program_id(1)
    t = xbf_ref.shape[0]
    ws = [w_ref[i].astype(jnp.bfloat16) for i in range(4)]

    ct = 1024
    for tb in range(t // ct):
        sl = pl.ds(tb * ct, ct)
        x = xbf_ref[sl, :]
        comb = comb_ref[sl, :]
        lane = jax.lax.broadcasted_iota(jnp.int32, (ct, NE), 1)
        upd = None
        for i in range(4):
            ci = jnp.sum(jnp.where(lane == 4 * q + i, comb, 0.0), axis=1,
                         keepdims=True)
            yi = jax.lax.dot_general(
                x, ws[i], (((1,), (1,)), ((), ())),
                preferred_element_type=jnp.float32,
            )
            upd = ci * yi if upd is None else upd + ci * yi

        @pl.when(q == 0)
        def _init(upd=upd, comb=comb, sl=sl):
            bias = jax.lax.dot_general(
                comb, eb_ref[...], (((1,), (0,)), ((), ())),
                precision=jax.lax.Precision.HIGHEST,
                preferred_element_type=jnp.float32,
            )
            out_ref[sl, :] = bias + upd

        @pl.when(q != 0)
        def _acc(upd=upd, sl=sl):
            out_ref[sl, :] += upd


def kernel(x, gate_w, gate_b, expert_w, expert_b):
    bsz, seq, h = x.shape
    t = bsz * seq
    xf = x.reshape(t, h)

    bt = 1024
    router = pl.pallas_call(
        _router_body,
        grid=(t // bt,),
        in_specs=[
            pl.BlockSpec((bt, h), lambda i: (i, 0)),
            pl.BlockSpec((NE, h), lambda i: (0, 0)),
            pl.BlockSpec((1, NE), lambda i: (0, 0)),
        ],
        out_specs=[
            pl.BlockSpec((bt, NE), lambda i: (i, 0)),
            pl.BlockSpec((bt, TOPK), lambda i: (i, 0)),
            pl.BlockSpec((bt, NE), lambda i: (i, 0)),
            pl.BlockSpec((bt, h), lambda i: (i, 0)),
        ],
        out_shape=[
            jax.ShapeDtypeStruct((t, NE), jnp.float32),
            jax.ShapeDtypeStruct((t, TOPK), jnp.int32),
            jax.ShapeDtypeStruct((t, NE), jnp.float32),
            jax.ShapeDtypeStruct((t, h), jnp.bfloat16),
        ],
    )
    probs, topk_idx, combine, x_bf = router(xf, gate_w, gate_b.reshape(1, NE))

    bh = 256
    experts = pl.pallas_call(
        _expert_body,
        grid=(h // bh, NE // 4),
        in_specs=[
            pl.BlockSpec((t, h), lambda j, p: (0, 0)),
            pl.BlockSpec((4, bh, h), lambda j, p: (p, j, 0)),
            pl.BlockSpec((t, NE), lambda j, p: (0, 0)),
            pl.BlockSpec((NE, bh), lambda j, p: (0, j)),
        ],
        out_specs=pl.BlockSpec((t, bh), lambda j, p: (0, j)),
        out_shape=jax.ShapeDtypeStruct((t, h), jnp.float32),
        compiler_params=pltpu.CompilerParams(
            dimension_semantics=("parallel", "arbitrary"),
        ),
    )
    out = experts(x_bf, expert_w, combine, expert_b)
    return (out.reshape(bsz, seq, h), topk_idx, probs)
